# baseline (speedup 1.0000x reference)
import functools

import jax
import jax.numpy as jnp
from jax import lax
from jax.experimental import pallas as pl
from jax.experimental.pallas import tpu as pltpu
from jax.experimental.pallas import tpu_sc as plsc

V = 100000
H = 128
B = 1024
S = 50
N = B * S

_NC = 2
_NS = 16
NW = _NC * _NS
ROWS_PER_W = N // NW
CB = 80
NCHUNK = ROWS_PER_W // CB


def _sc_gather(emb, idx3):
    mesh = plsc.VectorSubcoreMesh(core_axis_name="c", subcore_axis_name="s")

    @functools.partial(
        pl.kernel,
        out_type=jax.ShapeDtypeStruct((NW, NCHUNK, CB, H), jnp.float32),
        mesh=mesh,
        scratch_types=[
            pltpu.VMEM((NCHUNK, CB), jnp.int32),
            pltpu.VMEM((2, CB, H), jnp.float32),
            pltpu.SemaphoreType.DMA,
            pltpu.SemaphoreType.DMA,
            pltpu.SemaphoreType.DMA,
            pltpu.SemaphoreType.DMA,
        ],
    )
    def k(emb_hbm, idx_hbm, out_hbm, idx_v, rows_v, g0, g1, o0, o1):
        wid = lax.axis_index("s") * _NC + lax.axis_index("c")
        pltpu.sync_copy(idx_hbm.at[wid], idx_v)
        gsems = (g0, g1)
        osems = (o0, o1)

        gathers = [None, None]
        outs = [None, None]
        gathers[0] = pltpu.async_copy(emb_hbm.at[idx_v.at[0]], rows_v.at[0], g0)
        for j in range(NCHUNK):
            b = j % 2
            nb = (j + 1) % 2
            if j + 1 < NCHUNK:
                if outs[nb] is not None:
                    outs[nb].wait()
                    outs[nb] = None
                gathers[nb] = pltpu.async_copy(
                    emb_hbm.at[idx_v.at[j + 1]], rows_v.at[nb], gsems[nb])
            gathers[b].wait()
            outs[b] = pltpu.async_copy(rows_v.at[b], out_hbm.at[wid, j], osems[b])
        outs[(NCHUNK - 1) % 2].wait()
        if outs[NCHUNK % 2] is not None:
            outs[NCHUNK % 2].wait()

    return k(emb, idx3)


TSTEPS = 8
NG = (S + TSTEPS - 1) // TSTEPS


def _gru_body(x_ref, wih_ref, whh_ref, bih_ref, bhh_ref, out_ref, h_ref):
    g = pl.program_id(0)

    @pl.when(g == 0)
    def _():
        h_ref[...] = jnp.zeros_like(h_ref)

    h = h_ref[...]
    wih = wih_ref[...]
    whh = whh_ref[...]
    for k in range(TSTEPS):
        x = x_ref[k]
        gi = jnp.dot(x, wih, preferred_element_type=jnp.float32) + bih_ref[...]
        gh = jnp.dot(h, whh, preferred_element_type=jnp.float32) + bhh_ref[...]
        r = 0.5 + 0.5 * jnp.tanh(0.5 * (gi[:, :H] + gh[:, :H]))
        z = 0.5 + 0.5 * jnp.tanh(0.5 * (gi[:, H:2 * H] + gh[:, H:2 * H]))
        n = jnp.tanh(gi[:, 2 * H:] + r * gh[:, 2 * H:])
        h = n + z * (h - n)
        out_ref[:, k, :] = h
    h_ref[...] = h


def _gru(x_sbh, wih_t, whh_t, bih2, bhh2):
    return pl.pallas_call(
        _gru_body,
        grid=(NG,),
        in_specs=[
            pl.BlockSpec((TSTEPS, B, H), lambda t: (t, 0, 0)),
            pl.BlockSpec((H, 3 * H), lambda t: (0, 0)),
            pl.BlockSpec((H, 3 * H), lambda t: (0, 0)),
            pl.BlockSpec((1, 3 * H), lambda t: (0, 0)),
            pl.BlockSpec((1, 3 * H), lambda t: (0, 0)),
        ],
        out_specs=pl.BlockSpec((B, TSTEPS, H), lambda t: (0, t, 0)),
        out_shape=jax.ShapeDtypeStruct((B, S, H), jnp.float32),
        scratch_shapes=[pltpu.VMEM((B, H), jnp.float32)],
        compiler_params=pltpu.CompilerParams(
            dimension_semantics=("arbitrary",)),
    )(x_sbh, wih_t, whh_t, bih2, bhh2)


def kernel(input, emb, W_ih, W_hh, b_ih, b_hh):
    idx3 = input.astype(jnp.int32).T.reshape(NW, NCHUNK, CB)
    x = _sc_gather(emb, idx3)
    x_sbh = x.reshape(S, B, H)
    enc = _gru(x_sbh, W_ih.T, W_hh.T,
               b_ih.reshape(1, 3 * H), b_hh.reshape(1, 3 * H))
    hidden = enc[:, S - 1, :][None]
    return (enc, hidden)


# speedup vs baseline: 1.2557x; 1.2557x over previous
import functools

import jax
import jax.numpy as jnp
from jax import lax
from jax.experimental import pallas as pl
from jax.experimental.pallas import tpu as pltpu
from jax.experimental.pallas import tpu_sc as plsc

V = 100000
H = 128
B = 1024
S = 50
SA = 30
SB = S - SA

_NC = 2
_NS = 16
NW = _NC * _NS
CB = 80

TSTEPS = 10


def _sc_gather(emb, idx3, nchunk):
    mesh = plsc.VectorSubcoreMesh(core_axis_name="c", subcore_axis_name="s")

    @functools.partial(
        pl.kernel,
        out_type=jax.ShapeDtypeStruct((NW, nchunk, CB, H), jnp.float32),
        mesh=mesh,
        scratch_types=[
            pltpu.VMEM((nchunk, CB), jnp.int32),
            pltpu.VMEM((2, CB, H), jnp.float32),
            pltpu.SemaphoreType.DMA,
            pltpu.SemaphoreType.DMA,
            pltpu.SemaphoreType.DMA,
            pltpu.SemaphoreType.DMA,
        ],
    )
    def k(emb_hbm, idx_hbm, out_hbm, idx_v, rows_v, g0, g1, o0, o1):
        wid = lax.axis_index("s") * _NC + lax.axis_index("c")
        pltpu.sync_copy(idx_hbm.at[wid], idx_v)
        gsems = (g0, g1)
        osems = (o0, o1)

        gathers = [None, None]
        outs = [None, None]
        gathers[0] = pltpu.async_copy(emb_hbm.at[idx_v.at[0]], rows_v.at[0], g0)
        for j in range(nchunk):
            b = j % 2
            nb = (j + 1) % 2
            if j + 1 < nchunk:
                if outs[nb] is not None:
                    outs[nb].wait()
                    outs[nb] = None
                gathers[nb] = pltpu.async_copy(
                    emb_hbm.at[idx_v.at[j + 1]], rows_v.at[nb], gsems[nb])
            gathers[b].wait()
            outs[b] = pltpu.async_copy(rows_v.at[b], out_hbm.at[wid, j], osems[b])
        outs[(nchunk - 1) % 2].wait()
        if outs[nchunk % 2] is not None:
            outs[nchunk % 2].wait()

    return k(emb, idx3)


def _gru_step(x, h, wih, whh, bih, bhh):
    gi = jnp.dot(x, wih, preferred_element_type=jnp.float32) + bih
    gh = jnp.dot(h, whh, preferred_element_type=jnp.float32) + bhh
    r = 0.5 + 0.5 * jnp.tanh(0.5 * (gi[:, :H] + gh[:, :H]))
    z = 0.5 + 0.5 * jnp.tanh(0.5 * (gi[:, H:2 * H] + gh[:, H:2 * H]))
    n = jnp.tanh(gi[:, 2 * H:] + r * gh[:, 2 * H:])
    return n + z * (h - n)


def _gru_body_a(x_ref, wih_ref, whh_ref, bih_ref, bhh_ref, out_ref, hout_ref,
                h_ref):
    g = pl.program_id(0)

    @pl.when(g == 0)
    def _():
        h_ref[...] = jnp.zeros_like(h_ref)

    h = h_ref[...]
    wih = wih_ref[...]
    whh = whh_ref[...]
    for k in range(TSTEPS):
        h = _gru_step(x_ref[k], h, wih, whh, bih_ref[...], bhh_ref[...])
        out_ref[k] = h
    h_ref[...] = h
    hout_ref[...] = h


def _gru_body_b(x_ref, h0_ref, wih_ref, whh_ref, bih_ref, bhh_ref, out_ref,
                h_ref):
    g = pl.program_id(0)

    @pl.when(g == 0)
    def _():
        h_ref[...] = h0_ref[...]

    h = h_ref[...]
    wih = wih_ref[...]
    whh = whh_ref[...]
    for k in range(TSTEPS):
        h = _gru_step(x_ref[k], h, wih, whh, bih_ref[...], bhh_ref[...])
        out_ref[k] = h
    h_ref[...] = h


_W_SPEC = [
    pl.BlockSpec((H, 3 * H), lambda t: (0, 0)),
    pl.BlockSpec((H, 3 * H), lambda t: (0, 0)),
    pl.BlockSpec((1, 3 * H), lambda t: (0, 0)),
    pl.BlockSpec((1, 3 * H), lambda t: (0, 0)),
]


def _gru_a(x_sbh, wih_t, whh_t, bih2, bhh2):
    return pl.pallas_call(
        _gru_body_a,
        grid=(SA // TSTEPS,),
        in_specs=[pl.BlockSpec((TSTEPS, B, H), lambda t: (t, 0, 0))] + _W_SPEC,
        out_specs=[
            pl.BlockSpec((TSTEPS, B, H), lambda t: (t, 0, 0)),
            pl.BlockSpec((B, H), lambda t: (0, 0)),
        ],
        out_shape=[
            jax.ShapeDtypeStruct((SA, B, H), jnp.float32),
            jax.ShapeDtypeStruct((B, H), jnp.float32),
        ],
        scratch_shapes=[pltpu.VMEM((B, H), jnp.float32)],
        compiler_params=pltpu.CompilerParams(
            dimension_semantics=("arbitrary",)),
    )(x_sbh, wih_t, whh_t, bih2, bhh2)


def _gru_b(x_sbh, h0, wih_t, whh_t, bih2, bhh2):
    return pl.pallas_call(
        _gru_body_b,
        grid=(SB // TSTEPS,),
        in_specs=[pl.BlockSpec((TSTEPS, B, H), lambda t: (t, 0, 0)),
                  pl.BlockSpec((B, H), lambda t: (0, 0))] + _W_SPEC,
        out_specs=pl.BlockSpec((TSTEPS, B, H), lambda t: (t, 0, 0)),
        out_shape=jax.ShapeDtypeStruct((SB, B, H), jnp.float32),
        scratch_shapes=[pltpu.VMEM((B, H), jnp.float32)],
        compiler_params=pltpu.CompilerParams(
            dimension_semantics=("arbitrary",)),
    )(x_sbh, h0, wih_t, whh_t, bih2, bhh2)


def kernel(input, emb, W_ih, W_hh, b_ih, b_hh):
    idxT = input.astype(jnp.int32).T
    idx_a = idxT[:SA].reshape(NW, (SA * B) // (NW * CB), CB)
    idx_b = idxT[SA:].reshape(NW, (SB * B) // (NW * CB), CB)
    wih_t = W_ih.T
    whh_t = W_hh.T
    bih2 = b_ih.reshape(1, 3 * H)
    bhh2 = b_hh.reshape(1, 3 * H)

    xa = _sc_gather(emb, idx_a, (SA * B) // (NW * CB)).reshape(SA, B, H)
    xb = _sc_gather(emb, idx_b, (SB * B) // (NW * CB)).reshape(SB, B, H)

    enc_a, h_mid = _gru_a(xa, wih_t, whh_t, bih2, bhh2)
    enc_b = _gru_b(xb, h_mid, wih_t, whh_t, bih2, bhh2)

    enc = jnp.concatenate(
        [jnp.transpose(enc_a, (1, 0, 2)), jnp.transpose(enc_b, (1, 0, 2))],
        axis=1)
    hidden = enc_b[SB - 1][None]
    return (enc, hidden)


# speedup vs baseline: 1.5601x; 1.2424x over previous
import functools

import jax
import jax.numpy as jnp
from jax import lax
from jax.experimental import pallas as pl
from jax.experimental.pallas import tpu as pltpu
from jax.experimental.pallas import tpu_sc as plsc

V = 100000
H = 128
B = 1024
S = 50
N = B * S

_NC = 2
_NS = 16
NW = _NC * _NS
ROWS_PER_W = N // NW
CB = 80
NCHUNK = ROWS_PER_W // CB


def _sc_gather(emb, idx3):
    mesh = plsc.VectorSubcoreMesh(core_axis_name="c", subcore_axis_name="s")

    @functools.partial(
        pl.kernel,
        out_type=jax.ShapeDtypeStruct((NW, NCHUNK, CB, H), jnp.float32),
        mesh=mesh,
        scratch_types=[
            pltpu.VMEM((NCHUNK, CB), jnp.int32),
            pltpu.VMEM((2, CB, H), jnp.float32),
            pltpu.SemaphoreType.DMA,
            pltpu.SemaphoreType.DMA,
            pltpu.SemaphoreType.DMA,
            pltpu.SemaphoreType.DMA,
        ],
    )
    def k(emb_hbm, idx_hbm, out_hbm, idx_v, rows_v, g0, g1, o0, o1):
        wid = lax.axis_index("s") * _NC + lax.axis_index("c")
        pltpu.sync_copy(idx_hbm.at[wid], idx_v)
        gsems = (g0, g1)
        osems = (o0, o1)

        gathers = [None, None]
        outs = [None, None]
        gathers[0] = pltpu.async_copy(emb_hbm.at[idx_v.at[0]], rows_v.at[0], g0)
        for j in range(NCHUNK):
            b = j % 2
            nb = (j + 1) % 2
            if j + 1 < NCHUNK:
                if outs[nb] is not None:
                    outs[nb].wait()
                    outs[nb] = None
                gathers[nb] = pltpu.async_copy(
                    emb_hbm.at[idx_v.at[j + 1]], rows_v.at[nb], gsems[nb])
            gathers[b].wait()
            outs[b] = pltpu.async_copy(rows_v.at[b], out_hbm.at[wid, j], osems[b])
        outs[(NCHUNK - 1) % 2].wait()
        if outs[NCHUNK % 2] is not None:
            outs[NCHUNK % 2].wait()

    return k(emb, idx3)


TSTEPS = 10


def _gru_body(x_ref, wih_ref, whh_ref, bih_ref, bhh_ref, out_ref, h_ref):
    g = pl.program_id(0)

    @pl.when(g == 0)
    def _():
        h_ref[...] = jnp.zeros_like(h_ref)

    wih = wih_ref[...]
    whh = whh_ref[...]
    bih = bih_ref[...]
    bhh = bhh_ref[...]
    Bh = B // 2

    def step(x, h):
        gi = jnp.dot(x, wih, preferred_element_type=jnp.float32) + bih
        gh = jnp.dot(h, whh, preferred_element_type=jnp.float32) + bhh
        r = 0.5 + 0.5 * jnp.tanh(0.5 * (gi[:, :H] + gh[:, :H]))
        z = 0.5 + 0.5 * jnp.tanh(0.5 * (gi[:, H:2 * H] + gh[:, H:2 * H]))
        n = jnp.tanh(gi[:, 2 * H:] + r * gh[:, 2 * H:])
        return n + z * (h - n)

    h1 = h_ref[:Bh, :]
    h2 = h_ref[Bh:, :]
    for k in range(TSTEPS):
        h1 = step(x_ref[k, :Bh, :], h1)
        h2 = step(x_ref[k, Bh:, :], h2)
        out_ref[k, :Bh, :] = h1
        out_ref[k, Bh:, :] = h2
    h_ref[:Bh, :] = h1
    h_ref[Bh:, :] = h2


def _gru(x_sbh, wih_t, whh_t, bih2, bhh2):
    return pl.pallas_call(
        _gru_body,
        grid=(S // TSTEPS,),
        in_specs=[
            pl.BlockSpec((TSTEPS, B, H), lambda t: (t, 0, 0)),
            pl.BlockSpec((H, 3 * H), lambda t: (0, 0)),
            pl.BlockSpec((H, 3 * H), lambda t: (0, 0)),
            pl.BlockSpec((1, 3 * H), lambda t: (0, 0)),
            pl.BlockSpec((1, 3 * H), lambda t: (0, 0)),
        ],
        out_specs=pl.BlockSpec((TSTEPS, B, H), lambda t: (t, 0, 0)),
        out_shape=jax.ShapeDtypeStruct((S, B, H), jnp.float32),
        scratch_shapes=[pltpu.VMEM((B, H), jnp.float32)],
        compiler_params=pltpu.CompilerParams(
            dimension_semantics=("arbitrary",)),
    )(x_sbh, wih_t, whh_t, bih2, bhh2)


def kernel(input, emb, W_ih, W_hh, b_ih, b_hh):
    idx3 = input.astype(jnp.int32).T.reshape(NW, NCHUNK, CB)
    x = _sc_gather(emb, idx3)
    x_sbh = x.reshape(S, B, H)
    enc_t = _gru(x_sbh, W_ih.T, W_hh.T,
                 b_ih.reshape(1, 3 * H), b_hh.reshape(1, 3 * H))
    enc = jnp.transpose(enc_t, (1, 0, 2))
    hidden = enc_t[S - 1][None]
    return (enc, hidden)


# speedup vs baseline: 1.6092x; 1.0315x over previous
import functools

import jax
import jax.numpy as jnp
from jax import lax
from jax.experimental import pallas as pl
from jax.experimental.pallas import tpu as pltpu
from jax.experimental.pallas import tpu_sc as plsc

V = 100000
H = 128
B = 1024
S = 50
N = B * S

_NC = 2
_NS = 16
NW = _NC * _NS
ROWS_PER_W = N // NW
CB = 80
NCHUNK = ROWS_PER_W // CB


def _sc_gather(emb, idx3):
    mesh = plsc.VectorSubcoreMesh(core_axis_name="c", subcore_axis_name="s")

    @functools.partial(
        pl.kernel,
        out_type=jax.ShapeDtypeStruct((NW, NCHUNK, CB, H), jnp.float32),
        mesh=mesh,
        scratch_types=[
            pltpu.VMEM((NCHUNK, CB), jnp.int32),
            pltpu.VMEM((3, CB, H), jnp.float32),
            pltpu.SemaphoreType.DMA,
            pltpu.SemaphoreType.DMA,
            pltpu.SemaphoreType.DMA,
            pltpu.SemaphoreType.DMA,
            pltpu.SemaphoreType.DMA,
            pltpu.SemaphoreType.DMA,
        ],
    )
    def k(emb_hbm, idx_hbm, out_hbm, idx_v, rows_v, g0, g1, g2, o0, o1, o2):
        wid = lax.axis_index("s") * _NC + lax.axis_index("c")
        pltpu.sync_copy(idx_hbm.at[wid], idx_v)
        gsems = (g0, g1, g2)
        osems = (o0, o1, o2)

        gathers = [None, None, None]
        outs = [None, None, None]
        gathers[0] = pltpu.async_copy(emb_hbm.at[idx_v.at[0]], rows_v.at[0], g0)
        gathers[1] = pltpu.async_copy(emb_hbm.at[idx_v.at[1]], rows_v.at[1], g1)
        for j in range(NCHUNK):
            b = j % 3
            if j + 2 < NCHUNK:
                nb = (j + 2) % 3
                if outs[nb] is not None:
                    outs[nb].wait()
                    outs[nb] = None
                gathers[nb] = pltpu.async_copy(
                    emb_hbm.at[idx_v.at[j + 2]], rows_v.at[nb], gsems[nb])
            gathers[b].wait()
            outs[b] = pltpu.async_copy(rows_v.at[b], out_hbm.at[wid, j], osems[b])
        for b in range(3):
            if outs[b] is not None:
                outs[b].wait()

    return k(emb, idx3)


TSTEPS = 10


def _gru_body(x_ref, wih_ref, whh_ref, bih_ref, bhh_ref, out_ref, h_ref):
    g = pl.program_id(0)

    @pl.when(g == 0)
    def _():
        h_ref[...] = jnp.zeros_like(h_ref)

    wih = wih_ref[...]
    whh = whh_ref[...]
    bih = bih_ref[...]
    bhh = bhh_ref[...]
    Bh = B // 2

    def step(x, h):
        gi = jnp.dot(x, wih, preferred_element_type=jnp.float32) + bih
        gh = jnp.dot(h, whh, preferred_element_type=jnp.float32) + bhh
        r = 0.5 + 0.5 * jnp.tanh(0.5 * (gi[:, :H] + gh[:, :H]))
        z = 0.5 + 0.5 * jnp.tanh(0.5 * (gi[:, H:2 * H] + gh[:, H:2 * H]))
        n = jnp.tanh(gi[:, 2 * H:] + r * gh[:, 2 * H:])
        return n + z * (h - n)

    h1 = h_ref[:Bh, :]
    h2 = h_ref[Bh:, :]
    for k in range(TSTEPS):
        h1 = step(x_ref[k, :Bh, :], h1)
        h2 = step(x_ref[k, Bh:, :], h2)
        out_ref[k, :Bh, :] = h1
        out_ref[k, Bh:, :] = h2
    h_ref[:Bh, :] = h1
    h_ref[Bh:, :] = h2


def _gru(x_sbh, wih_t, whh_t, bih2, bhh2):
    return pl.pallas_call(
        _gru_body,
        grid=(S // TSTEPS,),
        in_specs=[
            pl.BlockSpec((TSTEPS, B, H), lambda t: (t, 0, 0)),
            pl.BlockSpec((H, 3 * H), lambda t: (0, 0)),
            pl.BlockSpec((H, 3 * H), lambda t: (0, 0)),
            pl.BlockSpec((1, 3 * H), lambda t: (0, 0)),
            pl.BlockSpec((1, 3 * H), lambda t: (0, 0)),
        ],
        out_specs=pl.BlockSpec((TSTEPS, B, H), lambda t: (t, 0, 0)),
        out_shape=jax.ShapeDtypeStruct((S, B, H), jnp.float32),
        scratch_shapes=[pltpu.VMEM((B, H), jnp.float32)],
        compiler_params=pltpu.CompilerParams(
            dimension_semantics=("arbitrary",)),
    )(x_sbh, wih_t, whh_t, bih2, bhh2)


def kernel(input, emb, W_ih, W_hh, b_ih, b_hh):
    idx3 = input.astype(jnp.int32).T.reshape(NW, NCHUNK, CB)
    x = _sc_gather(emb, idx3)
    x_sbh = x.reshape(S, B, H)
    enc_t = _gru(x_sbh, W_ih.T, W_hh.T,
                 b_ih.reshape(1, 3 * H), b_hh.reshape(1, 3 * H))
    enc = jnp.transpose(enc_t, (1, 0, 2))
    hidden = enc_t[S - 1][None]
    return (enc, hidden)
